# speedup vs baseline: 4.2556x; 4.2556x over previous
import functools

import jax
import jax.numpy as jnp
from jax.experimental import pallas as pl
from jax.experimental.pallas import tpu as pltpu

_B_BLK = 2048
_M_PAD = 1024


def _tc_body(z_ref, et_ref, out_ref, acc_ref, *, m_true):
    i = pl.program_id(0)
    nblk = pl.num_programs(0)

    @pl.when(i == 0)
    def _init():
        acc_ref[...] = jnp.full(acc_ref.shape, jnp.inf, dtype=jnp.float32)

    z = z_ref[...]
    et = et_ref[...]
    g = jax.lax.dot_general(
        z, et, (((1,), (0,)), ((), ())),
        preferred_element_type=jnp.float32,
        precision=jax.lax.Precision.HIGHEST,
    )
    znorm = jnp.sum(z * z, axis=1, keepdims=True)
    d = znorm - 2.0 * g
    d8 = jnp.min(d.reshape(_B_BLK // 8, 8, _M_PAD), axis=0)
    acc_ref[...] = jnp.minimum(acc_ref[...], d8)

    @pl.when(i == nblk - 1)
    def _fin():
        et_f = et_ref[...]
        enorm = jnp.sum(et_f * et_f, axis=0, keepdims=True)
        m = jnp.min(acc_ref[...], axis=0, keepdims=True) + enorm
        col = jax.lax.broadcasted_iota(jnp.int32, (1, _M_PAD), 1)
        s = jnp.sum(jnp.where(col < m_true, m, 0.0))
        out_ref[0, 0] = s / float(m_true)


def kernel(z, e, M):
    del M
    b, zd = z.shape
    m_true = e.shape[0]
    et = jnp.pad(e, ((0, _M_PAD - m_true), (0, 0))).T
    body = functools.partial(_tc_body, m_true=m_true)
    out = pl.pallas_call(
        body,
        grid=(b // _B_BLK,),
        in_specs=[
            pl.BlockSpec((_B_BLK, zd), lambda i: (i, 0)),
            pl.BlockSpec((zd, _M_PAD), lambda i: (0, 0)),
        ],
        out_specs=pl.BlockSpec(memory_space=pltpu.SMEM),
        out_shape=jax.ShapeDtypeStruct((1, 1), jnp.float32),
        scratch_shapes=[pltpu.VMEM((8, _M_PAD), jnp.float32)],
    )(z, et)
    return out.reshape(())
